# Initial kernel scaffold; baseline (speedup 1.0000x reference)
import jax
import jax.numpy as jnp
from jax.experimental import pallas as pl


def kernel(hidden_states, Wg, W1, W2):
    raise NotImplementedError("write your pallas kernel here")


# speedup vs baseline: 1.0560x; 1.0560x over previous
import functools

import jax
import jax.numpy as jnp
from jax import lax
from jax.experimental import pallas as pl
from jax.experimental.pallas import tpu as pltpu
from jax.experimental.pallas import tpu_sc as plsc

_E = 64
_D = 768
_DFF = 768
_T = 2048
_C = 64
_NW = 32
_TPW = _T // _NW
_EO_ROWS = (_E + 1) * _C


def _gate_body(x_ref, wg_ref, sx_ref, slot_ref):
    x = x_ref[...]
    logits = jnp.dot(x, wg_ref[...], preferred_element_type=jnp.float32)
    m = jnp.max(logits, axis=1, keepdims=True)
    e = jnp.exp(logits - m)
    p = e / jnp.sum(e, axis=1, keepdims=True)
    gate = jnp.max(p, axis=1, keepdims=True)
    lane = lax.broadcasted_iota(jnp.int32, p.shape, 1)
    idx = jnp.min(jnp.where(p == gate, lane, _E), axis=1, keepdims=True)
    onehot = (lane == idx).astype(jnp.float32)

    nb = _T // 128
    oh3 = onehot.astype(jnp.bfloat16).reshape(nb, 128, _E)
    r = lax.broadcasted_iota(jnp.int32, (128, 128), 0)
    c = lax.broadcasted_iota(jnp.int32, (128, 128), 1)
    tril = (r >= c).astype(jnp.bfloat16)
    totals = jnp.sum(onehot.reshape(nb, 128, _E), axis=1)
    rb = lax.broadcasted_iota(jnp.int32, (nb, nb), 0)
    cb = lax.broadcasted_iota(jnp.int32, (nb, nb), 1)
    stril = (rb > cb).astype(jnp.float32)
    offs = jnp.dot(stril, totals, preferred_element_type=jnp.float32,
                   precision=lax.Precision.HIGHEST)
    blocks = []
    for b in range(nb):
        incl = jnp.dot(tril, oh3[b], preferred_element_type=jnp.float32)
        blocks.append(incl + offs[b:b + 1, :])
    locations = jnp.concatenate(blocks, axis=0) - 1.0

    keepm = onehot * (locations < float(_C)).astype(jnp.float32)
    kept = jnp.sum(keepm, axis=1, keepdims=True)
    loc1 = jnp.sum(locations * keepm, axis=1, keepdims=True)
    slot_f = idx.astype(jnp.float32) * float(_C) + loc1
    slot_ref[...] = jnp.where(kept > 0, slot_f, float(_E * _C)).astype(jnp.int32)
    sx_ref[...] = x * (gate * kept)


def _gate(tokens, Wg):
    return pl.pallas_call(
        _gate_body,
        out_shape=(jax.ShapeDtypeStruct((_T, _D), jnp.float32),
                   jax.ShapeDtypeStruct((_T, 1), jnp.int32)),
    )(tokens, Wg)


def _mlp_body(disp_ref, w1_ref, w2_ref, out_ref):
    e = pl.program_id(0)

    @pl.when(e < _E)
    def _():
        x = disp_ref[...]
        h = jnp.maximum(
            jnp.dot(x, w1_ref[0], preferred_element_type=jnp.float32,
                    precision=lax.Precision.DEFAULT), 0.0)
        out_ref[...] = jnp.dot(h, w2_ref[0], preferred_element_type=jnp.float32,
                               precision=lax.Precision.DEFAULT)

    @pl.when(e == _E)
    def _():
        out_ref[...] = jnp.zeros_like(out_ref)


def _mlp(disp, W1, W2):
    return pl.pallas_call(
        _mlp_body,
        grid=(_E + 1,),
        in_specs=[
            pl.BlockSpec((_C, _D), lambda e: (e, 0)),
            pl.BlockSpec((1, _D, _DFF), lambda e: (jnp.minimum(e, _E - 1), 0, 0)),
            pl.BlockSpec((1, _DFF, _D), lambda e: (jnp.minimum(e, _E - 1), 0, 0)),
        ],
        out_specs=pl.BlockSpec((_C, _D), lambda e: (e, 0)),
        out_shape=jax.ShapeDtypeStruct((_EO_ROWS, _D), jnp.float32),
    )(disp, W1, W2)


@functools.cache
def _sc_kernels():
    mesh = plsc.VectorSubcoreMesh(core_axis_name="c", subcore_axis_name="s")

    @functools.partial(
        pl.kernel,
        out_type=jax.ShapeDtypeStruct((_EO_ROWS, _D), jnp.float32),
        mesh=mesh,
        scratch_types=[pltpu.VMEM((_TPW,), jnp.int32),
                       pltpu.VMEM((_TPW, _D), jnp.float32),
                       pltpu.SemaphoreType.DMA],
    )
    def dispatch_sc(sx_hbm, slot_hbm, disp_hbm, idx_v, rows_v, sem):
        wid = lax.axis_index("s") * 2 + lax.axis_index("c")
        base = wid * _TPW
        pltpu.sync_copy(slot_hbm.at[wid], idx_v)
        pltpu.sync_copy(sx_hbm.at[pl.ds(base, _TPW)], rows_v)
        pltpu.async_copy(rows_v, disp_hbm.at[idx_v], sem).wait()

    @functools.partial(
        pl.kernel,
        out_type=jax.ShapeDtypeStruct((_T, _D), jnp.float32),
        mesh=mesh,
        scratch_types=[pltpu.VMEM((_TPW,), jnp.int32),
                       pltpu.VMEM((_TPW, _D), jnp.float32),
                       pltpu.SemaphoreType.DMA],
    )
    def combine_sc(eo_hbm, slot_hbm, out_hbm, idx_v, rows_v, sem):
        wid = lax.axis_index("s") * 2 + lax.axis_index("c")
        base = wid * _TPW
        pltpu.sync_copy(slot_hbm.at[wid], idx_v)
        pltpu.async_copy(eo_hbm.at[idx_v], rows_v, sem).wait()
        pltpu.sync_copy(rows_v, out_hbm.at[pl.ds(base, _TPW)])

    return dispatch_sc, combine_sc


def kernel(hidden_states, Wg, W1, W2):
    B, S, D = hidden_states.shape
    tokens = jnp.transpose(hidden_states, (1, 0, 2)).reshape(S * B, D)
    dispatch_sc, combine_sc = _sc_kernels()
    sx, slot = _gate(tokens, Wg)
    slot_w = slot.reshape(_NW, _TPW)
    disp = dispatch_sc(sx, slot_w)
    eo = _mlp(disp, W1, W2)
    out_tok = combine_sc(eo, slot_w)
    return jnp.transpose(out_tok.reshape(S, B, D), (1, 0, 2))
